# baseline (speedup 1.0000x reference)
import functools

import jax
import jax.numpy as jnp
from jax import lax
from jax.experimental import pallas as pl
from jax.experimental.pallas import tpu as pltpu
from jax.experimental.pallas import tpu_sc as plsc

N_NODES = 10000
N_EDGES = 320000
D = 128

NC = 2
NS = 16
NW = NC * NS
E_PER_W = N_EDGES // NW
CHUNK = 128
NCH = 80
E_PAD_W = NCH * CHUNK
N_ACC = N_NODES + NS
ROWS_PER_TILE = 624
TAIL_OFF = ROWS_PER_TILE * NS
TAIL_ROWS = N_ACC - TAIL_OFF
WB_TAIL = N_NODES - TAIL_OFF


def _sc_scatter_sum(features, packed_idx, zeros):
    mesh = plsc.VectorSubcoreMesh(
        core_axis_name="c", subcore_axis_name="s", num_cores=NC, num_subcores=NS
    )

    @functools.partial(
        pl.kernel,
        out_type=jax.ShapeDtypeStruct((NC, N_NODES, D), jnp.float32),
        mesh=mesh,
        scratch_types=[
            pltpu.VMEM_SHARED((N_ACC, D), jnp.float32),
            pltpu.VMEM((2, CHUNK), jnp.int32),
            pltpu.VMEM((2, CHUNK), jnp.int32),
            pltpu.VMEM((CHUNK, D), jnp.float32),
            pltpu.VMEM((CHUNK, D), jnp.float32),
            pltpu.SemaphoreType.DMA,
            pltpu.SemaphoreType.DMA,
            pltpu.SemaphoreType.DMA,
            pltpu.SemaphoreType.DMA,
        ],
    )
    def k(feat_hbm, idx_hbm, zeros_hbm, out_hbm,
          acc, idx0, idx1, rows0, rows1, sg0, sg1, ss0, ss1):
        c = lax.axis_index("c")
        s = lax.axis_index("s")
        wid = s * NC + c

        pltpu.sync_copy(zeros_hbm, acc.at[pl.ds(s * ROWS_PER_TILE, ROWS_PER_TILE)])

        @pl.when(s == NS - 1)
        def _():
            pltpu.sync_copy(zeros_hbm.at[pl.ds(0, TAIL_ROWS)],
                            acc.at[pl.ds(TAIL_OFF, TAIL_ROWS)])

        plsc.subcore_barrier()

        pltpu.sync_copy(idx_hbm.at[wid, 0], idx0)
        pltpu.async_copy(feat_hbm.at[idx0.at[0]], rows0, sg0)
        pltpu.sync_copy(idx_hbm.at[wid, 1], idx1)
        pltpu.async_copy(feat_hbm.at[idx1.at[0]], rows1, sg1)

        def body(g, _):
            j0 = 2 * g
            pltpu.make_async_copy(feat_hbm.at[idx0.at[0]], rows0, sg0).wait()
            pltpu.async_copy(rows0, acc.at[idx0.at[1]], ss0, add=True)
            pltpu.make_async_copy(feat_hbm.at[idx1.at[0]], rows1, sg1).wait()
            pltpu.async_copy(rows1, acc.at[idx1.at[1]], ss1, add=True)

            @pl.when(j0 + 2 < NCH)
            def _():
                pltpu.make_async_copy(rows0, acc.at[idx0.at[1]], ss0).wait()
                pltpu.sync_copy(idx_hbm.at[wid, j0 + 2], idx0)
                pltpu.async_copy(feat_hbm.at[idx0.at[0]], rows0, sg0)
                pltpu.make_async_copy(rows1, acc.at[idx1.at[1]], ss1).wait()
                pltpu.sync_copy(idx_hbm.at[wid, j0 + 3], idx1)
                pltpu.async_copy(feat_hbm.at[idx1.at[0]], rows1, sg1)

            return ()

        lax.fori_loop(0, NCH // 2, body, ())

        pltpu.make_async_copy(rows0, acc.at[idx0.at[1]], ss0).wait()
        pltpu.make_async_copy(rows1, acc.at[idx1.at[1]], ss1).wait()

        plsc.subcore_barrier()
        pltpu.sync_copy(
            acc.at[pl.ds(s * ROWS_PER_TILE, ROWS_PER_TILE)],
            out_hbm.at[c, pl.ds(s * ROWS_PER_TILE, ROWS_PER_TILE)],
        )

        @pl.when(s == NS - 1)
        def _():
            pltpu.sync_copy(acc.at[pl.ds(TAIL_OFF, WB_TAIL)],
                            out_hbm.at[c, pl.ds(TAIL_OFF, WB_TAIL)])

    return k(features, packed_idx, zeros)


def _tc_body(f_ref, p0_ref, p1_ref, wt_ref, b_ref, t_ref, o_ref):
    t1 = t_ref[0, 0]
    t2 = t_ref[0, 1]
    wt = wt_ref[...]
    b = b_ref[...]
    a1 = jnp.dot(f_ref[...], wt, preferred_element_type=jnp.float32) + b
    hn = p0_ref[...] + p1_ref[...]
    a2 = jnp.dot(hn, wt, preferred_element_type=jnp.float32) + b
    o_ref[...] = t1 * jnp.maximum(a1, 0.0) + t2 * jnp.maximum(a2, 0.0)


def _tc_combine(features, partials, W, b, theta1, theta2):
    wt = W.T
    b2 = b.reshape(1, D)
    thetas = jnp.stack([theta1[0], theta2[0]]).reshape(1, 2)
    R = 1000
    grid = (N_NODES // R,)
    return pl.pallas_call(
        _tc_body,
        grid=grid,
        in_specs=[
            pl.BlockSpec((R, D), lambda i: (i, 0)),
            pl.BlockSpec((R, D), lambda i: (i, 0)),
            pl.BlockSpec((R, D), lambda i: (i, 0)),
            pl.BlockSpec((D, D), lambda i: (0, 0)),
            pl.BlockSpec((1, D), lambda i: (0, 0)),
            pl.BlockSpec(memory_space=pltpu.SMEM),
        ],
        out_specs=pl.BlockSpec((R, D), lambda i: (i, 0)),
        out_shape=jax.ShapeDtypeStruct((N_NODES, D), jnp.float32),
    )(features, partials[0], partials[1], wt, b2, thetas)


@jax.jit
def kernel(features, edge_index, W, b, theta1, theta2):
    src = edge_index[0].astype(jnp.int32).reshape(NW, E_PER_W)
    dst = edge_index[1].astype(jnp.int32).reshape(NW, E_PER_W)
    pad = E_PAD_W - E_PER_W
    src_p = jnp.pad(src, ((0, 0), (0, pad)))
    trash = (N_NODES + jnp.arange(NW, dtype=jnp.int32) // NC)[:, None]
    dst_p = jnp.concatenate(
        [dst, jnp.broadcast_to(trash, (NW, pad))], axis=1)
    packed = jnp.stack(
        [src_p.reshape(NW, NCH, CHUNK), dst_p.reshape(NW, NCH, CHUNK)], axis=2
    )
    zeros = jnp.zeros((ROWS_PER_TILE, D), jnp.float32)
    partials = _sc_scatter_sum(features, packed, zeros)
    return _tc_combine(features, partials, W, b, theta1, theta2)


# speedup vs baseline: 1.0156x; 1.0156x over previous
import functools

import jax
import jax.numpy as jnp
from jax import lax
from jax.experimental import pallas as pl
from jax.experimental.pallas import tpu as pltpu
from jax.experimental.pallas import tpu_sc as plsc

N_NODES = 10000
N_EDGES = 320000
D = 128

NC = 2
NS = 16
NW = NC * NS
E_PER_W = N_EDGES // NW
CHUNK = 128
NCH = 80
E_PAD_W = NCH * CHUNK
N_ACC = N_NODES + NS
ROWS_PER_TILE = 624
TAIL_OFF = ROWS_PER_TILE * NS
TAIL_ROWS = N_ACC - TAIL_OFF
WB_TAIL = N_NODES - TAIL_OFF


def _sc_scatter_sum(features, packed_idx, zeros):
    mesh = plsc.VectorSubcoreMesh(
        core_axis_name="c", subcore_axis_name="s", num_cores=NC, num_subcores=NS
    )

    @functools.partial(
        pl.kernel,
        out_type=jax.ShapeDtypeStruct((NC, N_NODES, D), jnp.float32),
        mesh=mesh,
        scratch_types=[
            pltpu.VMEM_SHARED((N_ACC, D), jnp.float32),
            pltpu.VMEM((2, CHUNK), jnp.int32),
            pltpu.VMEM((2, CHUNK), jnp.int32),
            pltpu.VMEM((CHUNK, D), jnp.float32),
            pltpu.VMEM((CHUNK, D), jnp.float32),
            pltpu.SemaphoreType.DMA,
            pltpu.SemaphoreType.DMA,
        ],
    )
    def k(feat_hbm, idx_hbm, zeros_hbm, out_hbm,
          acc, idx0, idx1, rows0, rows1, sg0, sg1):
        c = lax.axis_index("c")
        s = lax.axis_index("s")
        wid = s * NC + c

        pltpu.sync_copy(zeros_hbm, acc.at[pl.ds(s * ROWS_PER_TILE, ROWS_PER_TILE)])

        @pl.when(s == NS - 1)
        def _():
            pltpu.sync_copy(zeros_hbm.at[pl.ds(0, TAIL_ROWS)],
                            acc.at[pl.ds(TAIL_OFF, TAIL_ROWS)])

        plsc.subcore_barrier()

        pltpu.sync_copy(idx_hbm.at[wid, 0], idx0)
        pltpu.async_copy(feat_hbm.at[idx0.at[0]], rows0, sg0)

        def body(g, _):
            j0 = 2 * g
            pltpu.sync_copy(idx_hbm.at[wid, j0 + 1], idx1)
            pltpu.async_copy(feat_hbm.at[idx1.at[0]], rows1, sg1)
            pltpu.make_async_copy(feat_hbm.at[idx0.at[0]], rows0, sg0).wait()
            pltpu.sync_copy(rows0, acc.at[idx0.at[1]], add=True)

            @pl.when(j0 + 2 < NCH)
            def _():
                pltpu.sync_copy(idx_hbm.at[wid, j0 + 2], idx0)
                pltpu.async_copy(feat_hbm.at[idx0.at[0]], rows0, sg0)

            pltpu.make_async_copy(feat_hbm.at[idx1.at[0]], rows1, sg1).wait()
            pltpu.sync_copy(rows1, acc.at[idx1.at[1]], add=True)
            return ()

        lax.fori_loop(0, NCH // 2, body, ())

        plsc.subcore_barrier()
        pltpu.sync_copy(
            acc.at[pl.ds(s * ROWS_PER_TILE, ROWS_PER_TILE)],
            out_hbm.at[c, pl.ds(s * ROWS_PER_TILE, ROWS_PER_TILE)],
        )

        @pl.when(s == NS - 1)
        def _():
            pltpu.sync_copy(acc.at[pl.ds(TAIL_OFF, WB_TAIL)],
                            out_hbm.at[c, pl.ds(TAIL_OFF, WB_TAIL)])

    return k(features, packed_idx, zeros)


def _tc_body(f_ref, p0_ref, p1_ref, wt_ref, b_ref, t_ref, o_ref):
    t1 = t_ref[0, 0]
    t2 = t_ref[0, 1]
    wt = wt_ref[...]
    b = b_ref[...]
    a1 = jnp.dot(f_ref[...], wt, preferred_element_type=jnp.float32) + b
    hn = p0_ref[...] + p1_ref[...]
    a2 = jnp.dot(hn, wt, preferred_element_type=jnp.float32) + b
    o_ref[...] = t1 * jnp.maximum(a1, 0.0) + t2 * jnp.maximum(a2, 0.0)


def _tc_combine(features, partials, W, b, theta1, theta2):
    wt = W.T
    b2 = b.reshape(1, D)
    thetas = jnp.stack([theta1[0], theta2[0]]).reshape(1, 2)
    R = 1000
    grid = (N_NODES // R,)
    return pl.pallas_call(
        _tc_body,
        grid=grid,
        in_specs=[
            pl.BlockSpec((R, D), lambda i: (i, 0)),
            pl.BlockSpec((R, D), lambda i: (i, 0)),
            pl.BlockSpec((R, D), lambda i: (i, 0)),
            pl.BlockSpec((D, D), lambda i: (0, 0)),
            pl.BlockSpec((1, D), lambda i: (0, 0)),
            pl.BlockSpec(memory_space=pltpu.SMEM),
        ],
        out_specs=pl.BlockSpec((R, D), lambda i: (i, 0)),
        out_shape=jax.ShapeDtypeStruct((N_NODES, D), jnp.float32),
    )(features, partials[0], partials[1], wt, b2, thetas)


@jax.jit
def kernel(features, edge_index, W, b, theta1, theta2):
    src = edge_index[0].astype(jnp.int32).reshape(NW, E_PER_W)
    dst = edge_index[1].astype(jnp.int32).reshape(NW, E_PER_W)
    pad = E_PAD_W - E_PER_W
    src_p = jnp.pad(src, ((0, 0), (0, pad)))
    trash = (N_NODES + jnp.arange(NW, dtype=jnp.int32) // NC)[:, None]
    dst_p = jnp.concatenate(
        [dst, jnp.broadcast_to(trash, (NW, pad))], axis=1)
    packed = jnp.stack(
        [src_p.reshape(NW, NCH, CHUNK), dst_p.reshape(NW, NCH, CHUNK)], axis=2
    )
    zeros = jnp.zeros((ROWS_PER_TILE, D), jnp.float32)
    partials = _sc_scatter_sum(features, packed, zeros)
    return _tc_combine(features, partials, W, b, theta1, theta2)
